# baseline (speedup 1.0000x reference)
import functools

import jax
import jax.numpy as jnp
from jax import lax
from jax.experimental import pallas as pl
from jax.experimental.pallas import tpu as pltpu
from jax.experimental.pallas import tpu_sc as plsc

NV = 100000
NE = 50000
H = 128
NINC = 3 * NE

NUM_CORES = 2
NUM_SUBCORES = 16
LANES = 16

PER_TILE = 9728
NINC_PAD = PER_TILE * NUM_SUBCORES
CHUNK = 128
RAW = 1216
NRAW = PER_TILE // RAW
CAP = RAW + CHUNK
RNG = 11250
ZROWS = 125
NZ = RNG // ZROWS


def _mm1_body(x_ref, w_ref, b_ref, wt_ref, y_ref):
    x = x_ref[...]
    y = jnp.dot(x, w_ref[...], preferred_element_type=jnp.float32) + b_ref[...]
    y_ref[...] = jnp.maximum(y, 0.0) * wt_ref[...]


def _linear(v, W, b, wt, blk=1000):
    n = v.shape[0]
    return pl.pallas_call(
        _mm1_body,
        grid=(n // blk,),
        in_specs=[
            pl.BlockSpec((blk, H), lambda i: (i, 0)),
            pl.BlockSpec((H, H), lambda i: (0, 0)),
            pl.BlockSpec((1, H), lambda i: (0, 0)),
            pl.BlockSpec((blk, 1), lambda i: (i, 0)),
        ],
        out_specs=pl.BlockSpec((blk, H), lambda i: (i, 0)),
        out_shape=jax.ShapeDtypeStruct((n, H), jnp.float32),
    )(v, W, b.reshape(1, H), wt)


def _stage3_body(e_ref, sc_ref, ers_ref, w_ref, b_ref, ew_ref, enew_ref, evw_ref):
    e_new = (e_ref[...] + sc_ref[...]) / ers_ref[...]
    enew_ref[...] = e_new
    y = jnp.dot(e_new, w_ref[...], preferred_element_type=jnp.float32) + b_ref[...]
    evw_ref[...] = jnp.maximum(y, 0.0) * ew_ref[...]


def _stage3(e, scat, ers, W, b, ew, blk=1000):
    n = e.shape[0]
    return pl.pallas_call(
        _stage3_body,
        grid=(n // blk,),
        in_specs=[
            pl.BlockSpec((blk, H), lambda i: (i, 0)),
            pl.BlockSpec((blk, H), lambda i: (i, 0)),
            pl.BlockSpec((blk, 1), lambda i: (i, 0)),
            pl.BlockSpec((H, H), lambda i: (0, 0)),
            pl.BlockSpec((1, H), lambda i: (0, 0)),
            pl.BlockSpec((blk, 1), lambda i: (i, 0)),
        ],
        out_specs=[
            pl.BlockSpec((blk, H), lambda i: (i, 0)),
            pl.BlockSpec((blk, H), lambda i: (i, 0)),
        ],
        out_shape=[
            jax.ShapeDtypeStruct((n, H), jnp.float32),
            jax.ShapeDtypeStruct((n, H), jnp.float32),
        ],
    )(e, scat, ers, W, b.reshape(1, H), ew)


def _stage5_body(v_ref, vw_ref, sc_ref, vrs_ref, out_ref):
    out_ref[...] = (v_ref[...] * vw_ref[...] + sc_ref[...]) / vrs_ref[...]


def _stage5(v, vw, scat, vrs, blk=1000):
    n = v.shape[0]
    return pl.pallas_call(
        _stage5_body,
        grid=(n // blk,),
        in_specs=[
            pl.BlockSpec((blk, H), lambda i: (i, 0)),
            pl.BlockSpec((blk, 1), lambda i: (i, 0)),
            pl.BlockSpec((blk, H), lambda i: (i, 0)),
            pl.BlockSpec((blk, 1), lambda i: (i, 0)),
        ],
        out_specs=pl.BlockSpec((blk, H), lambda i: (i, 0)),
        out_shape=jax.ShapeDtypeStruct((n, H), jnp.float32),
    )(v, vw, scat, vrs)


def _bcast(w16, jj):
    return lax.gather(
        w16, jnp.full((LANES, 1), jj, jnp.int32),
        lax.GatherDimensionNumbers(
            offset_dims=(), collapsed_slice_dims=(0,), start_index_map=(0,)),
        (1,),
        mode=lax.GatherScatterMode.PROMISE_IN_BOUNDS)


def _make_sc_scatter(n_src, n_dst):
    nrng = -(-n_dst // RNG)
    nrng_sc = -(-nrng // NUM_CORES)
    mesh = plsc.VectorSubcoreMesh(
        core_axis_name="c", subcore_axis_name="s",
        num_cores=NUM_CORES, num_subcores=NUM_SUBCORES)

    @functools.partial(
        pl.kernel,
        mesh=mesh,
        compiler_params=pltpu.CompilerParams(
            use_tc_tiling_on_sc=False, needs_layout_passes=False),
        out_type=jax.ShapeDtypeStruct((n_dst, H), jnp.float32),
        scratch_types=[
            pltpu.VMEM((RAW,), jnp.int32),
            pltpu.VMEM((RAW,), jnp.int32),
            pltpu.VMEM((RAW,), jnp.float32),
            pltpu.VMEM((CAP,), jnp.int32),
            pltpu.VMEM((CAP,), jnp.int32),
            pltpu.VMEM((CAP,), jnp.float32),
            [pltpu.VMEM((CHUNK, H), jnp.float32) for _ in range(2)],
            [pltpu.VMEM((1, CHUNK), jnp.int32) for _ in range(2)],
            pltpu.VMEM_SHARED((RNG, H), jnp.float32),
            pltpu.SemaphoreType.DMA((2,)),
        ],
    )
    def k(tab, gidx, sidx, w, out, rg, rs, rw, cg, cs, cw, rowss, six2,
          acc, sem_g):
        c = lax.axis_index("c")
        s = lax.axis_index("s")
        base = s * PER_TILE

        z16 = jnp.zeros((LANES,), jnp.float32)

        def gather_copies(ch, rb):
            return [(tab.at[cg.at[pl.ds(ch * CHUNK, CHUNK)]], rowss[rb])]

        for j in range(nrng_sc):
            rng_i = c * nrng_sc + j
            lo = rng_i * RNG

            @pl.when(rng_i < nrng)
            def _():
                def zfill(r, _):
                    for l in range(H // LANES):
                        rowss[0][r, pl.ds(l * LANES, LANES)] = z16
                    return 0
                lax.fori_loop(0, ZROWS, zfill, 0)
                for z in range(-(-NZ // NUM_SUBCORES)):
                    u = s + z * NUM_SUBCORES
                    @pl.when(u < NZ)
                    def _():
                        pltpu.sync_copy(rowss[0].at[pl.ds(0, ZROWS)],
                                        acc.at[pl.ds(u * ZROWS, ZROWS)])
                plsc.subcore_barrier()

                def raw_chunk(rc, _):
                    rbase = base + rc * RAW
                    pltpu.sync_copy(gidx.at[pl.ds(rbase, RAW)], rg)
                    pltpu.sync_copy(sidx.at[pl.ds(rbase, RAW)], rs)
                    pltpu.sync_copy(w.at[pl.ds(rbase, RAW)], rw)

                    def binb(g, cur):
                        sl = pl.ds(g * LANES, LANES)
                        sx = rs[sl]
                        m = (sx >= lo) & (sx < lo + RNG)
                        cnt = plsc.all_reduce_population_count(m)
                        plsc.store_compressed(cs.at[pl.ds(cur, LANES)],
                                              sx - lo, mask=m)
                        plsc.store_compressed(cg.at[pl.ds(cur, LANES)],
                                              rg[sl], mask=m)
                        plsc.store_compressed(cw.at[pl.ds(cur, LANES)],
                                              rw[sl], mask=m)
                        return cur + cnt[0]
                    n_c = lax.fori_loop(0, RAW // LANES, binb, jnp.int32(0))

                    pad_g = lax.iota(jnp.int32, LANES) + s * LANES
                    zi = jnp.zeros((LANES,), jnp.int32)
                    for i in range(CHUNK // LANES):
                        psl = pl.ds(n_c + i * LANES, LANES)
                        cg[psl] = pad_g
                        cs[psl] = zi
                        cw[psl] = z16

                    nch = (n_c + CHUNK - 1) // CHUNK

                    @pl.when(nch > 0)
                    def _():
                        for src, dst in gather_copies(0, 0):
                            pltpu.async_copy(src, dst, sem_g.at[0])

                    def pipe(i2, _):
                        for off in range(2):
                            ch = i2 * 2 + off
                            rb = off
                            @pl.when(ch < nch)
                            def _():
                                @pl.when(ch + 1 < nch)
                                def _():
                                    for src, dst in gather_copies(ch + 1, 1 - rb):
                                        pltpu.async_copy(src, dst,
                                                         sem_g.at[1 - rb])
                                for src, dst in gather_copies(ch, rb):
                                    pltpu.make_async_copy(src, dst,
                                                          sem_g.at[rb]).wait()
                                def scale(g, _):
                                    w16 = cw[pl.ds(ch * CHUNK + g * LANES,
                                                   LANES)]
                                    for jj in range(LANES):
                                        bw = _bcast(w16, jj)
                                        r = g * LANES + jj
                                        for l in range(H // LANES):
                                            rsl = pl.ds(l * LANES, LANES)
                                            rowss[rb][r, rsl] = (
                                                rowss[rb][r, rsl] * bw)
                                    return 0
                                lax.fori_loop(0, CHUNK // LANES, scale, 0)
                                for i in range(CHUNK // LANES):
                                    six2[rb][0, pl.ds(i * LANES, LANES)] = (
                                        cs[pl.ds(ch * CHUNK + i * LANES,
                                                 LANES)])
                                pltpu.sync_copy(rowss[rb],
                                                acc.at[six2[rb].at[0]],
                                                add=True)
                        return 0
                    lax.fori_loop(0, (nch + 1) // 2, pipe, 0)
                    return 0
                lax.fori_loop(0, NRAW, raw_chunk, 0)
                plsc.subcore_barrier()

                for z in range(-(-NZ // NUM_SUBCORES)):
                    u = s + z * NUM_SUBCORES
                    @pl.when((u < NZ) & (lo + u * ZROWS < n_dst))
                    def _():
                        r0 = u * ZROWS
                        pltpu.sync_copy(acc.at[pl.ds(r0, ZROWS)],
                                        out.at[pl.ds(lo + r0, ZROWS)])
                plsc.subcore_barrier()

    return k


def kernel(v, e, batch_idx, W_v2e, W_e2v, b_v, b_e, paper_author, eidx, vidx,
           v_weight, e_weight, v_reg_weight, e_reg_weight, e_reg_sum, v_reg_sum):
    pad = NINC_PAD - NINC
    pad_i = jnp.arange(pad, dtype=jnp.int32)

    def prep(g, sidx_, wt, n_src, n_dst):
        g = jnp.concatenate([g.astype(jnp.int32), pad_i % n_src])
        si = jnp.concatenate([sidx_.astype(jnp.int32), pad_i % n_dst])
        w = jnp.concatenate([wt.reshape(-1), jnp.zeros((pad,), jnp.float32)])
        return g, si, w

    ve_w = _linear(v, W_v2e, b_v, v_weight)

    g_e, s_e, w_e = prep(paper_author[:, 0], eidx, v_reg_weight, NV, NE)
    scat_e = _make_sc_scatter(NV, NE)(ve_w, g_e, s_e, w_e)

    e_new, ev_w = _stage3(e, scat_e, e_reg_sum, W_e2v, b_e, e_weight)

    g_v, s_v, w_v = prep(paper_author[:, 1], vidx, e_reg_weight, NE, NV)
    scat_v = _make_sc_scatter(NE, NV)(ev_w, g_v, s_v, w_v)

    v_out = _stage5(v, v_weight, scat_v, v_reg_sum)
    return (v_out, e_new)


# speedup vs baseline: 1.1759x; 1.1759x over previous
import functools

import jax
import jax.numpy as jnp
from jax import lax
from jax.experimental import pallas as pl
from jax.experimental.pallas import tpu as pltpu
from jax.experimental.pallas import tpu_sc as plsc

NV = 100000
NE = 50000
H = 128
NINC = 3 * NE

NUM_CORES = 2
NUM_SUBCORES = 16
LANES = 16

PER_TILE = 9728
NINC_PAD = PER_TILE * NUM_SUBCORES
CHUNK = 64
RAW = 1216
NRAW = PER_TILE // RAW
CAP = RAW + CHUNK
RNG = 12500
ZROWS = 50
NZ = RNG // ZROWS


def _mm1_body(x_ref, w_ref, b_ref, wt_ref, y_ref):
    x = x_ref[...]
    y = jnp.dot(x, w_ref[...], preferred_element_type=jnp.float32) + b_ref[...]
    y_ref[...] = jnp.maximum(y, 0.0) * wt_ref[...]


def _linear(v, W, b, wt, blk=1000):
    n = v.shape[0]
    return pl.pallas_call(
        _mm1_body,
        grid=(n // blk,),
        in_specs=[
            pl.BlockSpec((blk, H), lambda i: (i, 0)),
            pl.BlockSpec((H, H), lambda i: (0, 0)),
            pl.BlockSpec((1, H), lambda i: (0, 0)),
            pl.BlockSpec((blk, 1), lambda i: (i, 0)),
        ],
        out_specs=pl.BlockSpec((blk, H), lambda i: (i, 0)),
        out_shape=jax.ShapeDtypeStruct((n, H), jnp.float32),
    )(v, W, b.reshape(1, H), wt)


def _stage3_body(e_ref, sc_ref, ers_ref, w_ref, b_ref, ew_ref, enew_ref, evw_ref):
    e_new = (e_ref[...] + sc_ref[...]) / ers_ref[...]
    enew_ref[...] = e_new
    y = jnp.dot(e_new, w_ref[...], preferred_element_type=jnp.float32) + b_ref[...]
    evw_ref[...] = jnp.maximum(y, 0.0) * ew_ref[...]


def _stage3(e, scat, ers, W, b, ew, blk=1000):
    n = e.shape[0]
    return pl.pallas_call(
        _stage3_body,
        grid=(n // blk,),
        in_specs=[
            pl.BlockSpec((blk, H), lambda i: (i, 0)),
            pl.BlockSpec((blk, H), lambda i: (i, 0)),
            pl.BlockSpec((blk, 1), lambda i: (i, 0)),
            pl.BlockSpec((H, H), lambda i: (0, 0)),
            pl.BlockSpec((1, H), lambda i: (0, 0)),
            pl.BlockSpec((blk, 1), lambda i: (i, 0)),
        ],
        out_specs=[
            pl.BlockSpec((blk, H), lambda i: (i, 0)),
            pl.BlockSpec((blk, H), lambda i: (i, 0)),
        ],
        out_shape=[
            jax.ShapeDtypeStruct((n, H), jnp.float32),
            jax.ShapeDtypeStruct((n, H), jnp.float32),
        ],
    )(e, scat, ers, W, b.reshape(1, H), ew)


def _stage5_body(v_ref, vw_ref, sc_ref, vrs_ref, out_ref):
    out_ref[...] = (v_ref[...] * vw_ref[...] + sc_ref[...]) / vrs_ref[...]


def _stage5(v, vw, scat, vrs, blk=1000):
    n = v.shape[0]
    return pl.pallas_call(
        _stage5_body,
        grid=(n // blk,),
        in_specs=[
            pl.BlockSpec((blk, H), lambda i: (i, 0)),
            pl.BlockSpec((blk, 1), lambda i: (i, 0)),
            pl.BlockSpec((blk, H), lambda i: (i, 0)),
            pl.BlockSpec((blk, 1), lambda i: (i, 0)),
        ],
        out_specs=pl.BlockSpec((blk, H), lambda i: (i, 0)),
        out_shape=jax.ShapeDtypeStruct((n, H), jnp.float32),
    )(v, vw, scat, vrs)


def _bcast(w16, jj):
    return lax.gather(
        w16, jnp.full((LANES, 1), jj, jnp.int32),
        lax.GatherDimensionNumbers(
            offset_dims=(), collapsed_slice_dims=(0,), start_index_map=(0,)),
        (1,),
        mode=lax.GatherScatterMode.PROMISE_IN_BOUNDS)


def _make_sc_scatter(n_src, n_dst):
    nrng = -(-n_dst // RNG)
    nrng_sc = -(-nrng // NUM_CORES)
    mesh = plsc.VectorSubcoreMesh(
        core_axis_name="c", subcore_axis_name="s",
        num_cores=NUM_CORES, num_subcores=NUM_SUBCORES)

    @functools.partial(
        pl.kernel,
        mesh=mesh,
        compiler_params=pltpu.CompilerParams(
            use_tc_tiling_on_sc=False, needs_layout_passes=False),
        out_type=jax.ShapeDtypeStruct((n_dst, H), jnp.float32),
        scratch_types=[
            pltpu.VMEM((RAW,), jnp.int32),
            pltpu.VMEM((RAW,), jnp.int32),
            pltpu.VMEM((RAW,), jnp.float32),
            pltpu.VMEM((CAP,), jnp.int32),
            pltpu.VMEM((CAP,), jnp.int32),
            pltpu.VMEM((CAP,), jnp.float32),
            [pltpu.VMEM((CHUNK, H), jnp.float32) for _ in range(2)],
            [pltpu.VMEM((1, CHUNK), jnp.int32) for _ in range(2)],
            pltpu.VMEM_SHARED((RNG, H), jnp.float32),
            pltpu.SemaphoreType.DMA((2,)),
        ],
    )
    def k(tab, gidx, sidx, w, out, rg, rs, rw, cg, cs, cw, rowss, six2,
          acc, sem_g):
        c = lax.axis_index("c")
        s = lax.axis_index("s")
        base = s * PER_TILE

        z16 = jnp.zeros((LANES,), jnp.float32)

        def gather_copies(ch, rb):
            return [(tab.at[cg.at[pl.ds(ch * CHUNK, CHUNK)]], rowss[rb])]

        for j in range(nrng_sc):
            rng_i = c * nrng_sc + j
            lo = rng_i * RNG

            @pl.when(rng_i < nrng)
            def _():
                def zfill(r, _):
                    for l in range(H // LANES):
                        rowss[0][r, pl.ds(l * LANES, LANES)] = z16
                    return 0
                lax.fori_loop(0, ZROWS, zfill, 0)
                for z in range(-(-NZ // NUM_SUBCORES)):
                    u = s + z * NUM_SUBCORES
                    @pl.when(u < NZ)
                    def _():
                        pltpu.sync_copy(rowss[0].at[pl.ds(0, ZROWS)],
                                        acc.at[pl.ds(u * ZROWS, ZROWS)])
                plsc.subcore_barrier()

                def raw_chunk(rc, _):
                    rbase = base + rc * RAW
                    pltpu.sync_copy(gidx.at[pl.ds(rbase, RAW)], rg)
                    pltpu.sync_copy(sidx.at[pl.ds(rbase, RAW)], rs)
                    pltpu.sync_copy(w.at[pl.ds(rbase, RAW)], rw)

                    def binb(g, cur):
                        sl = pl.ds(g * LANES, LANES)
                        sx = rs[sl]
                        m = (sx >= lo) & (sx < lo + RNG)
                        cnt = plsc.all_reduce_population_count(m)
                        plsc.store_compressed(cs.at[pl.ds(cur, LANES)],
                                              sx - lo, mask=m)
                        plsc.store_compressed(cg.at[pl.ds(cur, LANES)],
                                              rg[sl], mask=m)
                        plsc.store_compressed(cw.at[pl.ds(cur, LANES)],
                                              rw[sl], mask=m)
                        return cur + cnt[0]
                    n_c = lax.fori_loop(0, RAW // LANES, binb, jnp.int32(0))

                    pad_g = lax.iota(jnp.int32, LANES) + s * LANES
                    zi = jnp.zeros((LANES,), jnp.int32)
                    for i in range(CHUNK // LANES):
                        psl = pl.ds(n_c + i * LANES, LANES)
                        cg[psl] = pad_g
                        cs[psl] = zi
                        cw[psl] = z16

                    nch = (n_c + CHUNK - 1) // CHUNK

                    @pl.when(nch > 0)
                    def _():
                        for src, dst in gather_copies(0, 0):
                            pltpu.async_copy(src, dst, sem_g.at[0])

                    def pipe(i2, _):
                        for off in range(2):
                            ch = i2 * 2 + off
                            rb = off
                            @pl.when(ch < nch)
                            def _():
                                @pl.when(ch + 1 < nch)
                                def _():
                                    for src, dst in gather_copies(ch + 1, 1 - rb):
                                        pltpu.async_copy(src, dst,
                                                         sem_g.at[1 - rb])
                                for src, dst in gather_copies(ch, rb):
                                    pltpu.make_async_copy(src, dst,
                                                          sem_g.at[rb]).wait()
                                def scale(g, _):
                                    w16 = cw[pl.ds(ch * CHUNK + g * LANES,
                                                   LANES)]
                                    for jj in range(LANES):
                                        bw = _bcast(w16, jj)
                                        r = g * LANES + jj
                                        for l in range(H // LANES):
                                            rsl = pl.ds(l * LANES, LANES)
                                            rowss[rb][r, rsl] = (
                                                rowss[rb][r, rsl] * bw)
                                    return 0
                                lax.fori_loop(0, CHUNK // LANES, scale, 0)
                                for i in range(CHUNK // LANES):
                                    six2[rb][0, pl.ds(i * LANES, LANES)] = (
                                        cs[pl.ds(ch * CHUNK + i * LANES,
                                                 LANES)])
                                pltpu.sync_copy(rowss[rb],
                                                acc.at[six2[rb].at[0]],
                                                add=True)
                        return 0
                    lax.fori_loop(0, (nch + 1) // 2, pipe, 0)
                    return 0
                lax.fori_loop(0, NRAW, raw_chunk, 0)
                plsc.subcore_barrier()

                for z in range(-(-NZ // NUM_SUBCORES)):
                    u = s + z * NUM_SUBCORES
                    @pl.when((u < NZ) & (lo + u * ZROWS < n_dst))
                    def _():
                        r0 = u * ZROWS
                        pltpu.sync_copy(acc.at[pl.ds(r0, ZROWS)],
                                        out.at[pl.ds(lo + r0, ZROWS)])
                plsc.subcore_barrier()

    return k


def kernel(v, e, batch_idx, W_v2e, W_e2v, b_v, b_e, paper_author, eidx, vidx,
           v_weight, e_weight, v_reg_weight, e_reg_weight, e_reg_sum, v_reg_sum):
    pad = NINC_PAD - NINC
    pad_i = jnp.arange(pad, dtype=jnp.int32)

    def prep(g, sidx_, wt, n_src, n_dst):
        g = jnp.concatenate([g.astype(jnp.int32), pad_i % n_src])
        si = jnp.concatenate([sidx_.astype(jnp.int32), pad_i % n_dst])
        w = jnp.concatenate([wt.reshape(-1), jnp.zeros((pad,), jnp.float32)])
        return g, si, w

    ve_w = _linear(v, W_v2e, b_v, v_weight)

    g_e, s_e, w_e = prep(paper_author[:, 0], eidx, v_reg_weight, NV, NE)
    scat_e = _make_sc_scatter(NV, NE)(ve_w, g_e, s_e, w_e)

    e_new, ev_w = _stage3(e, scat_e, e_reg_sum, W_e2v, b_e, e_weight)

    g_v, s_v, w_v = prep(paper_author[:, 1], vidx, e_reg_weight, NE, NV)
    scat_v = _make_sc_scatter(NE, NV)(ev_w, g_v, s_v, w_v)

    v_out = _stage5(v, v_weight, scat_v, v_reg_sum)
    return (v_out, e_new)
